# baseline (speedup 1.0000x reference)
import functools

import jax
import jax.numpy as jnp
from jax import lax
from jax.experimental import pallas as pl
from jax.experimental.pallas import tpu as pltpu
from jax.experimental.pallas import tpu_sc as plsc

_ORDER = 16
_L = 16
_NC = 2
_NS = 16
_NW = _NC * _NS
_TOTAL = 16384 * 200
_PER_W = _TOTAL // _NW
_CHUNK = 25600
_NCHUNK = _PER_W // _CHUNK


def _body(inp_hbm, table_hbm, out_hbm, table_v, in_v, out_v):
    wid = lax.axis_index("s") * _NC + lax.axis_index("c")
    base = wid * _PER_W
    pltpu.sync_copy(table_hbm, table_v)

    def chunk_body(ci, carry):
        off = base + ci * _CHUNK
        pltpu.sync_copy(inp_hbm.at[pl.ds(off, _CHUNK)], in_v)

        def vec_body(i, c):
            idx = in_v[pl.ds(i * _L, _L)]
            out_v[pl.ds(i * _L, _L)] = plsc.load_gather(table_v, [idx])
            return c

        lax.fori_loop(0, _CHUNK // _L, vec_body, 0, unroll=4)
        pltpu.sync_copy(out_v, out_hbm.at[pl.ds(off, _CHUNK)])
        return carry

    lax.fori_loop(0, _NCHUNK, chunk_body, 0)


def kernel(input, val_table):
    flat = input.reshape(-1)
    mesh = plsc.VectorSubcoreMesh(core_axis_name="c", subcore_axis_name="s")
    run = pl.kernel(
        _body,
        mesh=mesh,
        out_type=jax.ShapeDtypeStruct((_TOTAL,), jnp.float32),
        scratch_types=[
            pltpu.VMEM((_ORDER,), jnp.float32),
            pltpu.VMEM((_CHUNK,), jnp.int32),
            pltpu.VMEM((_CHUNK,), jnp.float32),
        ],
        compiler_params=pltpu.CompilerParams(needs_layout_passes=False),
    )
    return run(flat, val_table).reshape(input.shape)


# speedup vs baseline: 1.4575x; 1.4575x over previous
import functools

import jax
import jax.numpy as jnp
from jax import lax
from jax.experimental import pallas as pl
from jax.experimental.pallas import tpu as pltpu
from jax.experimental.pallas import tpu_sc as plsc

_ORDER = 16
_L = 16
_NC = 2
_NS = 16
_NW = _NC * _NS
_TOTAL = 16384 * 200
_PER_W = _TOTAL // _NW
_CHUNK = 12800
_NCHUNK = _PER_W // _CHUNK
_NVEC = _CHUNK // _L


def _body(inp_hbm, table_hbm, out_hbm, table_v,
          in0, in1, out0, out1, si0, si1, so0, so1):
    wid = lax.axis_index("s") * _NC + lax.axis_index("c")
    base = wid * _PER_W
    pltpu.sync_copy(table_hbm, table_v)

    ins, outs = [in0, in1], [out0, out1]
    sin, sout = [si0, si1], [so0, so1]

    def start_in(ci):
        b = ci % 2
        return pltpu.async_copy(
            inp_hbm.at[pl.ds(base + ci * _CHUNK, _CHUNK)], ins[b], sin[b])

    def start_out(ci):
        b = ci % 2
        return pltpu.async_copy(
            outs[b], out_hbm.at[pl.ds(base + ci * _CHUNK, _CHUNK)], sout[b])

    in_copies = {0: start_in(0), 1: start_in(1)}
    out_copies = {}
    for ci in range(_NCHUNK):
        b = ci % 2
        in_copies[ci].wait()
        if ci >= 2:
            out_copies[ci - 2].wait()
        iv, ov = ins[b], outs[b]

        @plsc.parallel_loop(0, _NVEC, unroll=8)
        def _gather(i, iv=iv, ov=ov):
            idx = iv[pl.ds(i * _L, _L)]
            ov[pl.ds(i * _L, _L)] = plsc.load_gather(table_v, [idx])

        out_copies[ci] = start_out(ci)
        if ci + 2 < _NCHUNK:
            in_copies[ci + 2] = start_in(ci + 2)

    out_copies[_NCHUNK - 2].wait()
    out_copies[_NCHUNK - 1].wait()


def kernel(input, val_table):
    flat = input.reshape(-1)
    mesh = plsc.VectorSubcoreMesh(core_axis_name="c", subcore_axis_name="s")
    run = pl.kernel(
        _body,
        mesh=mesh,
        out_type=jax.ShapeDtypeStruct((_TOTAL,), jnp.float32),
        scratch_types=[
            pltpu.VMEM((_ORDER,), jnp.float32),
            pltpu.VMEM((_CHUNK,), jnp.int32),
            pltpu.VMEM((_CHUNK,), jnp.int32),
            pltpu.VMEM((_CHUNK,), jnp.float32),
            pltpu.VMEM((_CHUNK,), jnp.float32),
            pltpu.SemaphoreType.DMA,
            pltpu.SemaphoreType.DMA,
            pltpu.SemaphoreType.DMA,
            pltpu.SemaphoreType.DMA,
        ],
        compiler_params=pltpu.CompilerParams(needs_layout_passes=False),
    )
    return run(flat, val_table).reshape(input.shape)


# speedup vs baseline: 1.4833x; 1.0177x over previous
import functools

import jax
import jax.numpy as jnp
from jax import lax
from jax.experimental import pallas as pl
from jax.experimental.pallas import tpu as pltpu
from jax.experimental.pallas import tpu_sc as plsc

_ORDER = 16
_L = 16
_NC = 2
_NS = 16
_NW = _NC * _NS
_TOTAL = 16384 * 200
_PER_W = _TOTAL // _NW
_CHUNK = 12800
_NCHUNK = _PER_W // _CHUNK
_NVEC = _CHUNK // _L


def _body(inp_hbm, table_hbm, out_hbm, table_v,
          in0, in1, out0, out1, si0, si1, so0, so1):
    wid = lax.axis_index("s") * _NC + lax.axis_index("c")
    base = wid * _PER_W
    pltpu.sync_copy(table_hbm, table_v)
    tbl = table_v[...]

    ins, outs = [in0, in1], [out0, out1]
    sin, sout = [si0, si1], [so0, so1]

    def start_in(ci):
        b = ci % 2
        return pltpu.async_copy(
            inp_hbm.at[pl.ds(base + ci * _CHUNK, _CHUNK)], ins[b], sin[b])

    def start_out(ci):
        b = ci % 2
        return pltpu.async_copy(
            outs[b], out_hbm.at[pl.ds(base + ci * _CHUNK, _CHUNK)], sout[b])

    in_copies = {0: start_in(0), 1: start_in(1)}
    out_copies = {}
    for ci in range(_NCHUNK):
        b = ci % 2
        in_copies[ci].wait()
        if ci >= 2:
            out_copies[ci - 2].wait()
        iv, ov = ins[b], outs[b]

        @plsc.parallel_loop(0, _NVEC, unroll=8)
        def _gather(i, iv=iv, ov=ov):
            idx = iv[pl.ds(i * _L, _L)]
            ov[pl.ds(i * _L, _L)] = jnp.take_along_axis(
                tbl, idx, axis=0, mode="promise_in_bounds")

        out_copies[ci] = start_out(ci)
        if ci + 2 < _NCHUNK:
            in_copies[ci + 2] = start_in(ci + 2)

    out_copies[_NCHUNK - 2].wait()
    out_copies[_NCHUNK - 1].wait()


def kernel(input, val_table):
    flat = input.reshape(-1)
    mesh = plsc.VectorSubcoreMesh(core_axis_name="c", subcore_axis_name="s")
    run = pl.kernel(
        _body,
        mesh=mesh,
        out_type=jax.ShapeDtypeStruct((_TOTAL,), jnp.float32),
        scratch_types=[
            pltpu.VMEM((_ORDER,), jnp.float32),
            pltpu.VMEM((_CHUNK,), jnp.int32),
            pltpu.VMEM((_CHUNK,), jnp.int32),
            pltpu.VMEM((_CHUNK,), jnp.float32),
            pltpu.VMEM((_CHUNK,), jnp.float32),
            pltpu.SemaphoreType.DMA,
            pltpu.SemaphoreType.DMA,
            pltpu.SemaphoreType.DMA,
            pltpu.SemaphoreType.DMA,
        ],
        compiler_params=pltpu.CompilerParams(needs_layout_passes=False),
    )
    return run(flat, val_table).reshape(input.shape)


# speedup vs baseline: 5.1653x; 3.4823x over previous
import functools

import jax
import jax.numpy as jnp
from jax import lax
from jax.experimental import pallas as pl
from jax.experimental.pallas import tpu as pltpu
from jax.experimental.pallas import tpu_sc as plsc

_ORDER = 16
_L = 16
_NC = 2
_NS = 16
_NW = _NC * _NS
_ROWS = 200
_COLS = 16384
_CW = _COLS // _NW
_CR = 40
_NCHUNK = _ROWS // _CR
_NVEC = _CR * _CW // _L
_CVEC = _CW // _L


def _body(inp_hbm, table_hbm, out_hbm, table_v,
          in0, in1, out0, out1, si0, si1, so0, so1):
    wid = lax.axis_index("s") * _NC + lax.axis_index("c")
    col0 = wid * _CW
    pltpu.sync_copy(table_hbm, table_v)
    tbl = table_v[...]

    ins, outs = [in0, in1], [out0, out1]
    sin, sout = [si0, si1], [so0, so1]

    def start_in(ci):
        b = ci % 2
        return pltpu.async_copy(
            inp_hbm.at[pl.ds(ci * _CR, _CR), pl.ds(col0, _CW)], ins[b], sin[b])

    def start_out(ci):
        b = ci % 2
        return pltpu.async_copy(
            outs[b], out_hbm.at[pl.ds(ci * _CR, _CR), pl.ds(col0, _CW)],
            sout[b])

    in_copies = {0: start_in(0), 1: start_in(1)}
    out_copies = {}
    for ci in range(_NCHUNK):
        b = ci % 2
        in_copies[ci].wait()
        if ci >= 2:
            out_copies[ci - 2].wait()
        iv, ov = ins[b], outs[b]

        @plsc.parallel_loop(0, _NVEC, unroll=8)
        def _gather(i, iv=iv, ov=ov):
            r = i // _CVEC
            c = (i % _CVEC) * _L
            idx = iv[r, pl.ds(c, _L)]
            ov[r, pl.ds(c, _L)] = jnp.take_along_axis(
                tbl, idx, axis=0, mode="promise_in_bounds")

        out_copies[ci] = start_out(ci)
        if ci + 2 < _NCHUNK:
            in_copies[ci + 2] = start_in(ci + 2)

    for ci in range(max(0, _NCHUNK - 2), _NCHUNK):
        out_copies[ci].wait()


def kernel(input, val_table):
    xt = input.T
    mesh = plsc.VectorSubcoreMesh(core_axis_name="c", subcore_axis_name="s")
    run = pl.kernel(
        _body,
        mesh=mesh,
        out_type=jax.ShapeDtypeStruct((_ROWS, _COLS), jnp.float32),
        scratch_types=[
            pltpu.VMEM((_ORDER,), jnp.float32),
            pltpu.VMEM((_CR, _CW), jnp.int32),
            pltpu.VMEM((_CR, _CW), jnp.int32),
            pltpu.VMEM((_CR, _CW), jnp.float32),
            pltpu.VMEM((_CR, _CW), jnp.float32),
            pltpu.SemaphoreType.DMA,
            pltpu.SemaphoreType.DMA,
            pltpu.SemaphoreType.DMA,
            pltpu.SemaphoreType.DMA,
        ],
        compiler_params=pltpu.CompilerParams(
            needs_layout_passes=False, use_tc_tiling_on_sc=True),
    )
    return run(xt, val_table).T


# speedup vs baseline: 5.2015x; 1.0070x over previous
import functools

import jax
import jax.numpy as jnp
from jax import lax
from jax.experimental import pallas as pl
from jax.experimental.pallas import tpu as pltpu
from jax.experimental.pallas import tpu_sc as plsc

_ORDER = 16
_L = 16
_NC = 2
_NS = 16
_NW = _NC * _NS
_ROWS = 200
_COLS = 16384
_CW = _COLS // _NW
_CR = 40
_NCHUNK = _ROWS // _CR
_NVEC = _CR * _CW // _L
_CVEC = _CW // _L


def _body(inp_hbm, table_hbm, out_hbm, table_v,
          in0, in1, out0, out1, si0, si1, so0, so1):
    wid = lax.axis_index("s") * _NC + lax.axis_index("c")
    col0 = wid * _CW
    pltpu.sync_copy(table_hbm, table_v)
    tbl = table_v[...]

    ins, outs = [in0, in1], [out0, out1]
    sin, sout = [si0, si1], [so0, so1]

    def start_in(ci):
        b = ci % 2
        return pltpu.async_copy(
            inp_hbm.at[pl.ds(ci * _CR, _CR), pl.ds(col0, _CW)], ins[b], sin[b])

    def start_out(ci):
        b = ci % 2
        return pltpu.async_copy(
            outs[b], out_hbm.at[pl.ds(ci * _CR, _CR), pl.ds(col0, _CW)],
            sout[b])

    in_copies = {0: start_in(0), 1: start_in(1)}
    out_copies = {}
    for ci in range(_NCHUNK):
        b = ci % 2
        in_copies[ci].wait()
        if ci >= 2:
            out_copies[ci - 2].wait()
        iv, ov = ins[b], outs[b]

        @plsc.parallel_loop(0, _NVEC, unroll=8)
        def _gather(i, iv=iv, ov=ov):
            r = lax.shift_right_logical(i, 5)
            c = lax.shift_left(lax.bitwise_and(i, _CVEC - 1), 4)
            idx = iv[r, pl.ds(c, _L)]
            ov[r, pl.ds(c, _L)] = jnp.take_along_axis(
                tbl, idx, axis=0, mode="promise_in_bounds")

        out_copies[ci] = start_out(ci)
        if ci + 2 < _NCHUNK:
            in_copies[ci + 2] = start_in(ci + 2)

    for ci in range(max(0, _NCHUNK - 2), _NCHUNK):
        out_copies[ci].wait()


def kernel(input, val_table):
    xt = input.T
    mesh = plsc.VectorSubcoreMesh(core_axis_name="c", subcore_axis_name="s")
    run = pl.kernel(
        _body,
        mesh=mesh,
        out_type=jax.ShapeDtypeStruct((_ROWS, _COLS), jnp.float32),
        scratch_types=[
            pltpu.VMEM((_ORDER,), jnp.float32),
            pltpu.VMEM((_CR, _CW), jnp.int32),
            pltpu.VMEM((_CR, _CW), jnp.int32),
            pltpu.VMEM((_CR, _CW), jnp.float32),
            pltpu.VMEM((_CR, _CW), jnp.float32),
            pltpu.SemaphoreType.DMA,
            pltpu.SemaphoreType.DMA,
            pltpu.SemaphoreType.DMA,
            pltpu.SemaphoreType.DMA,
        ],
        compiler_params=pltpu.CompilerParams(
            needs_layout_passes=False, use_tc_tiling_on_sc=True),
    )
    return run(xt, val_table).T
